# baseline (speedup 1.0000x reference)
import math

import jax
import jax.numpy as jnp
import numpy as np
from jax import lax
from jax.experimental import pallas as pl
from jax.experimental.pallas import tpu as pltpu
from jax.experimental.pallas import tpu_sc as plsc

B = 1024
S = 50
V = 1000
NB = 100000
CAP = 30.0

NC = 2
NS = 16
NW = NC * NS
LANES = 16
BPW = B // NW
TPB = S - 1
TPW = BPW * TPB
K = 56
NCHUNK = TPW // K
NG = TPW // LANES
NFULL = V // LANES
TAIL_OFF = V - LANES
TAIL_NEW = V - NFULL * LANES

_F1 = np.float32(1.0)
_INV_CAP = np.float32(1.0 / CAP)
_CAPF = np.float32(CAP)


def _captanh(x):
    y = x * _INV_CAP
    t = jnp.exp(jnp.abs(y) * np.float32(-2.0))
    z = _CAPF * (_F1 - t) / (_F1 + t)
    return jnp.where(y < np.float32(0.0), -z, z)


def _sc_body(ids_hbm, tgt_hbm, w_hbm, out_a, out_s,
             ids_v, tgt_v, hidx_v, ttok_v, rows0, rows1,
             stage_zm, stage_sv, stage_a, stage_os, sem0, sem1):
    wid = lax.axis_index("s") * NC + lax.axis_index("c")
    b0 = wid * BPW
    lane = lax.broadcasted_iota(jnp.int32, (LANES,), 0)
    col0 = lane * 0

    pltpu.sync_copy(ids_hbm.at[pl.ds(b0, BPW), :], ids_v)
    pltpu.sync_copy(tgt_hbm.at[pl.ds(b0, BPW), :], tgt_v)

    def hash_body(_, kb):
        k16 = kb + lane
        r = lax.div(k16, np.int32(TPB))
        sm1 = lax.rem(k16, np.int32(TPB))
        s = sm1 + 1
        p1 = plsc.load_gather(ids_v, [r, s])
        p2 = plsc.load_gather(ids_v, [r, sm1])
        h = lax.rem((p2 * np.int32(1000003)) ^ (p1 * np.int32(92821)),
                    np.int32(NB))
        hidx_v[pl.ds(kb, LANES)] = h
        ttok_v[pl.ds(kb, LANES)] = plsc.load_gather(tgt_v, [r, s])
        return kb + np.int32(LANES)

    lax.fori_loop(0, NG, hash_body, np.int32(0))

    def _dma(c, rows, sem):
        return pltpu.make_async_copy(
            w_hbm.at[hidx_v.at[pl.ds(c * K, K)]], rows, sem)

    def _process(c, rows):
        def tok_body(_, i):
            def max_body(_, om):
                off, m = om
                m = jnp.maximum(m, rows[i, pl.ds(off, LANES)])
                return off + np.int32(LANES), m
            _, m_vec = lax.fori_loop(
                0, NFULL, max_body,
                (np.int32(0), jnp.full((LANES,), -np.inf, jnp.float32)))
            m_vec = jnp.maximum(m_vec, rows[i, pl.ds(TAIL_OFF, LANES)])
            zmax_vec = _captanh(jnp.full((LANES,), jnp.max(m_vec),
                                         jnp.float32))
            def sum_body(_, oa):
                off, acc = oa
                e = jnp.exp(_captanh(rows[i, pl.ds(off, LANES)])
                            - zmax_vec)
                return off + np.int32(LANES), acc + e
            _, acc = lax.fori_loop(0, NFULL, sum_body,
                                   (np.int32(0),
                                    jnp.zeros((LANES,), jnp.float32)))
            tail = jnp.exp(_captanh(rows[i, pl.ds(TAIL_OFF, LANES)])
                           - zmax_vec)
            tail = jnp.where(lane >= np.int32(LANES - TAIL_NEW), tail,
                             np.float32(0.0))
            s_tot = jnp.sum(acc + tail)
            stage_zm[i, :] = zmax_vec
            stage_sv[i, :] = jnp.full((LANES,), s_tot, jnp.float32)
            return i + np.int32(1)

        lax.fori_loop(0, K, tok_body, np.int32(0))

        for u in range(4):
            i16 = jnp.minimum(u * LANES + lane, np.int32(K - 1))
            t16 = ttok_v[pl.ds(c * K + u * LANES, LANES)]
            t16 = jnp.clip(t16, np.int32(0), np.int32(V - 1))
            z_t = _captanh(plsc.load_gather(rows, [i16, t16]))
            zm16 = plsc.load_gather(stage_zm, [i16, col0])
            s16 = plsc.load_gather(stage_sv, [i16, col0])
            stage_a[pl.ds(u * LANES, LANES)] = zm16 - z_t
            stage_os[pl.ds(u * LANES, LANES)] = s16

        pltpu.sync_copy(stage_a.at[pl.ds(0, K)], out_a.at[wid, c])
        pltpu.sync_copy(stage_os.at[pl.ds(0, K)], out_s.at[wid, c])

    _dma(0, rows0, sem0).start()

    def pair_body(_, c0):
        for bsel in range(2):
            rows, sem = (rows0, sem0) if bsel == 0 else (rows1, sem1)
            orows, osem = (rows1, sem1) if bsel == 0 else (rows0, sem0)
            c = c0 + np.int32(bsel)
            _dma(c, rows, sem).wait()
            if bsel == 0:
                _dma(c + np.int32(1), orows, osem).start()
            else:
                @pl.when(c < np.int32(NCHUNK - 1))
                def _():
                    _dma(c + np.int32(1), orows, osem).start()
            _process(c, rows)
        return c0 + np.int32(2)

    lax.fori_loop(0, NCHUNK // 2, pair_body, np.int32(0))


_sc_kernel = pl.kernel(
    _sc_body,
    out_type=(jax.ShapeDtypeStruct((NW, NCHUNK, K), jnp.float32),
              jax.ShapeDtypeStruct((NW, NCHUNK, K), jnp.float32)),
    mesh=plsc.VectorSubcoreMesh(core_axis_name="c", subcore_axis_name="s"),
    compiler_params=pltpu.CompilerParams(needs_layout_passes=False,
                                         use_tc_tiling_on_sc=False),
    scratch_types=[
        pltpu.VMEM((BPW, S), jnp.int32),
        pltpu.VMEM((BPW, S), jnp.int32),
        pltpu.VMEM((TPW,), jnp.int32),
        pltpu.VMEM((TPW + LANES,), jnp.int32),
        pltpu.VMEM((K, V), jnp.float32),
        pltpu.VMEM((K, V), jnp.float32),
        pltpu.VMEM((K, LANES), jnp.float32),
        pltpu.VMEM((K, LANES), jnp.float32),
        pltpu.VMEM((64,), jnp.float32),
        pltpu.VMEM((64,), jnp.float32),
        pltpu.SemaphoreType.DMA,
        pltpu.SemaphoreType.DMA,
    ],
)


def _finish_body(a_ref, s_ref, o_ref):
    nll_sum = jnp.sum(a_ref[...] + jnp.log(s_ref[...]))
    o_ref[0, 0] = (nll_sum + np.float32(B * math.log(V))) / np.float32(B * S)


_finish = pl.pallas_call(
    _finish_body,
    out_shape=jax.ShapeDtypeStruct((1, 1), jnp.float32),
    out_specs=pl.BlockSpec(memory_space=pltpu.SMEM),
)


def kernel(input_ids, target_ids, W):
    ids = input_ids.astype(jnp.int32)
    tgt = target_ids.astype(jnp.int32)
    W = W.astype(jnp.float32)
    out_a, out_s = _sc_kernel(ids, tgt, W)
    res = _finish(out_a.reshape(NW, NCHUNK * K), out_s.reshape(NW, NCHUNK * K))
    return res[0, 0]
